# Initial kernel scaffold; baseline (speedup 1.0000x reference)
import jax
import jax.numpy as jnp
from jax.experimental import pallas as pl


def kernel(x, edge_index, W1_self, W1_neigh, b1, W2_self, W2_neigh, b2):
    raise NotImplementedError("write your pallas kernel here")


# speedup vs baseline: 5.7834x; 5.7834x over previous
import functools
import jax
import jax.numpy as jnp
from jax import lax
from jax.experimental import pallas as pl
from jax.experimental.pallas import tpu as pltpu
from jax.experimental.pallas import tpu_sc as plsc

N = 10000
E = 320000
D_IN = 128
D_HID = 256
NCLS = 40

NC, NS, L = 2, 16, 16
NW = NC * NS
CH = 128
K = -(-E // (NW * CH))
EPAD = NW * K * CH
NPAD = 10112
RPT = NPAD // NS
DW1 = 144
DW2 = 48

@functools.lru_cache(maxsize=None)
def _make_seg_sum(D):
  mesh = plsc.VectorSubcoreMesh(core_axis_name="c", subcore_axis_name="s",
                                num_cores=NC, num_subcores=NS)

  @functools.partial(
      pl.kernel,
      out_type=jax.ShapeDtypeStruct((NC, NPAD, D), jnp.float32),
      mesh=mesh,
      scratch_types=[
          pltpu.VMEM((CH,), jnp.int32),
          pltpu.VMEM((CH,), jnp.int32),
          pltpu.VMEM((CH, D), jnp.float32),
          pltpu.SemaphoreType.DMA,
          pltpu.VMEM_SHARED((NPAD, D), jnp.float32),
      ],
      compiler_params=pltpu.CompilerParams(use_tc_tiling_on_sc=False),
  )
  def seg_sum(table_hbm, src_hbm, dst_hbm, out_hbm,
              src_v, dst_v, rows_v, sem, shared):
    c = lax.axis_index("c")
    s = lax.axis_index("s")
    wid = s * NC + c

    def zrow(r, _):
      def zcol(q, __):
        rows_v[r, pl.ds(q * L, L)] = jnp.zeros((L,), jnp.float32)
        return __
      return lax.fori_loop(0, D // L, zcol, _, unroll=True)
    lax.fori_loop(0, CH, zrow, 0)

    base = s * RPT
    for t in range(RPT // CH):
      pltpu.sync_copy(rows_v, shared.at[pl.ds(base + t * CH, CH)])
    rem = RPT % CH
    if rem:
      pltpu.sync_copy(rows_v.at[pl.ds(0, rem)],
                      shared.at[pl.ds(base + (RPT // CH) * CH, rem)])
    plsc.subcore_barrier()

    def chunk(j, _):
      pltpu.sync_copy(src_hbm.at[wid, j], src_v)
      pltpu.sync_copy(dst_hbm.at[wid, j], dst_v)
      pltpu.async_copy(table_hbm.at[src_v], rows_v, sem).wait()
      pltpu.sync_copy(rows_v, shared.at[dst_v], add=True)
      return _
    lax.fori_loop(0, K, chunk, 0)

    plsc.subcore_barrier()
    pltpu.sync_copy(shared.at[pl.ds(base, RPT)],
                    out_hbm.at[c, pl.ds(base, RPT)])

  return seg_sum


def _tc1_body(x_ref, aA_ref, aB_ref, dA_ref, dB_ref,
              w1s_ref, w1n_ref, b1_ref, w2n_ref, w2s_ref,
              p_ref, hs_ref):
  inv = 1.0 / jnp.maximum(dA_ref[:, :1] + dB_ref[:, :1], 1.0)
  hn = (aA_ref[...] + aB_ref[...]) * inv
  h = (jnp.dot(x_ref[...], w1s_ref[...], preferred_element_type=jnp.float32)
       + jnp.dot(hn, w1n_ref[...], preferred_element_type=jnp.float32)
       + b1_ref[...])
  h = jnp.maximum(h, 0.0)
  p_ref[...] = jnp.dot(h, w2n_ref[...], preferred_element_type=jnp.float32)
  hs_ref[...] = jnp.dot(h, w2s_ref[...], preferred_element_type=jnp.float32)


def _tc2_body(hs_ref, a2A_ref, a2B_ref, dA_ref, dB_ref, b2_ref, o_ref):
  inv = 1.0 / jnp.maximum(dA_ref[:, :1] + dB_ref[:, :1], 1.0)
  o_ref[...] = (hs_ref[...]
                + (a2A_ref[:, :NCLS] + a2B_ref[:, :NCLS]) * inv
                + b2_ref[...])


_R = 1000


def kernel(x, edge_index, W1_self, W1_neigh, b1, W2_self, W2_neigh, b2):
  f32 = jnp.float32
  src = edge_index[0].astype(jnp.int32)
  dst = edge_index[1].astype(jnp.int32)
  pad = EPAD - E
  src3 = jnp.concatenate([src, jnp.zeros((pad,), jnp.int32)]).reshape(NW, K, CH)
  dst3 = jnp.concatenate([dst, jnp.full((pad,), N, jnp.int32)]).reshape(NW, K, CH)

  xa = jnp.concatenate(
      [x, jnp.ones((N, 1), f32), jnp.zeros((N, DW1 - D_IN - 1), f32)], axis=1)

  agg1 = _make_seg_sum(DW1)(xa, src3, dst3)
  aA = agg1[0, :N, :D_IN]
  aB = agg1[1, :N, :D_IN]
  dA = agg1[0, :N, D_IN:D_IN + 8]
  dB = agg1[1, :N, D_IN:D_IN + 8]

  w1sT = W1_self.T
  w1nT = W1_neigh.T
  w2nT = jnp.pad(W2_neigh.T, ((0, 0), (0, DW2 - NCLS)))
  w2sT = W2_self.T
  b1r = b1.reshape(1, D_HID)
  b2r = b2.reshape(1, NCLS)

  grid = (N // _R,)
  row_spec = lambda w: pl.BlockSpec((_R, w), lambda i: (i, 0))
  full_spec = lambda a, b: pl.BlockSpec((a, b), lambda i: (0, 0))

  p, hs = pl.pallas_call(
      _tc1_body,
      grid=grid,
      in_specs=[
          row_spec(D_IN), row_spec(D_IN), row_spec(D_IN),
          row_spec(8), row_spec(8),
          full_spec(D_IN, D_HID), full_spec(D_IN, D_HID),
          full_spec(1, D_HID),
          full_spec(D_HID, DW2), full_spec(D_HID, NCLS),
      ],
      out_specs=[row_spec(DW2), row_spec(NCLS)],
      out_shape=[jax.ShapeDtypeStruct((N, DW2), f32),
                 jax.ShapeDtypeStruct((N, NCLS), f32)],
  )(x, aA, aB, dA, dB, w1sT, w1nT, b1r, w2nT, w2sT)

  agg2 = _make_seg_sum(DW2)(p, src3, dst3)

  out = pl.pallas_call(
      _tc2_body,
      grid=grid,
      in_specs=[
          row_spec(NCLS), row_spec(DW2), row_spec(DW2),
          row_spec(8), row_spec(8),
          full_spec(1, NCLS),
      ],
      out_specs=row_spec(NCLS),
      out_shape=jax.ShapeDtypeStruct((N, NCLS), f32),
  )(hs, agg2[0, :N], agg2[1, :N], dA, dB, b2r)

  return out
